# initial kernel scaffold (unmeasured)
import jax
import jax.numpy as jnp
from jax import lax
from jax.experimental import pallas as pl
from jax.experimental.pallas import tpu as pltpu

N_DEV = 4
BLK = 64
QT = 512
SCALE = 0.08838834764831843


def kernel(x, Wq, K_ext, V_ext, Wo):
    _, sq, dm = x.shape
    dm_, dq = Wq.shape
    _, skv, h_tot, dh = K_ext.shape
    hg = dq // dh
    n_qt = sq // QT

    def body(x_ref, wq_ref, k_ref, v_ref, wo_ref, out_ref,
             wq_all, wo_all, wq_cur, wo_cur, k_head, v_head,
             ctx_buf, bias_ref, copy_sems,
             wq_ssem, wq_rsem, wo_ssem, wo_rsem):
        s = pl.program_id(0)
        my = lax.axis_index("i")
        right = lax.rem(my + 1, N_DEV)
        left = lax.rem(my + N_DEV - 1, N_DEV)
        slot = lax.rem(my + N_DEV - s, N_DEV)
        prev_h = jnp.maximum(s - 1, 0)
        prev_send_slot = lax.rem(my + N_DEV + 1 - s, N_DEV)

        @pl.when(s == 0)
        def _init():
            barrier = pltpu.get_barrier_semaphore()
            for nbr in (left, right):
                pl.semaphore_signal(barrier, inc=1, device_id=(nbr,),
                                    device_id_type=pl.DeviceIdType.MESH)
            pl.semaphore_wait(barrier, 2)
            cp_q = pltpu.make_async_copy(wq_ref, wq_all.at[my], copy_sems.at[0])
            cp_o = pltpu.make_async_copy(wo_ref, wo_all.at[my], copy_sems.at[1])
            cp_q.start()
            cp_o.start()
            for qt in range(n_qt):
                r = lax.broadcasted_iota(jnp.int32, (QT, skv), 0)
                c = lax.broadcasted_iota(jnp.int32, (QT, skv), 1)
                qb = (my * sq + qt * QT + r) // BLK
                kb = c // BLK
                keep = (qb == kb) | (kb == 0) | (lax.rem(qb + kb, 3) == 0)
                bias_ref[qt * QT:(qt + 1) * QT, :] = jnp.where(
                    keep, 0.0, -1e9).astype(jnp.bfloat16)
            cp_q.wait()
            cp_o.wait()
            out_ref[...] = jnp.zeros(out_ref.shape, out_ref.dtype)

        @pl.when(s > 0)
        def _wait_prev_hop():
            for slab, ssem, rsem in ((wq_all, wq_ssem, wq_rsem),
                                     (wo_all, wo_ssem, wo_rsem)):
                pltpu.make_async_remote_copy(
                    src_ref=slab.at[prev_send_slot],
                    dst_ref=slab.at[slot],
                    send_sem=ssem.at[prev_h],
                    recv_sem=rsem.at[prev_h],
                    device_id=(right,),
                    device_id_type=pl.DeviceIdType.MESH,
                ).wait()

        @pl.when(s < N_DEV - 1)
        def _send_hop():
            for slab, ssem, rsem in ((wq_all, wq_ssem, wq_rsem),
                                     (wo_all, wo_ssem, wo_rsem)):
                pltpu.make_async_remote_copy(
                    src_ref=slab.at[slot],
                    dst_ref=slab.at[slot],
                    send_sem=ssem.at[s],
                    recv_sem=rsem.at[s],
                    device_id=(right,),
                    device_id_type=pl.DeviceIdType.MESH,
                ).start()

        ld_q = pltpu.make_async_copy(wq_all.at[slot], wq_cur, copy_sems.at[0])
        ld_o = pltpu.make_async_copy(wo_all.at[slot], wo_cur, copy_sems.at[1])
        ld_q.start()
        ld_o.start()
        ld_q.wait()
        ld_o.wait()

        head0 = slot * hg
        for qt in range(n_qt):
            rows = pl.ds(qt * QT, QT)
            q = jnp.dot(x_ref[0, rows, :], wq_cur[...],
                        preferred_element_type=jnp.float32) * SCALE
            bias = bias_ref[rows, :].astype(jnp.float32)
            for h in range(hg):
                ck = pltpu.make_async_copy(
                    k_ref.at[0, :, head0 + h, :], k_head, copy_sems.at[2])
                cv = pltpu.make_async_copy(
                    v_ref.at[0, :, head0 + h, :], v_head, copy_sems.at[3])
                ck.start()
                cv.start()
                ck.wait()
                cv.wait()
                sc = lax.dot_general(
                    q[:, h * dh:(h + 1) * dh], k_head[...],
                    (((1,), (1,)), ((), ())),
                    preferred_element_type=jnp.float32)
                p = jnp.exp(sc + bias)
                denom = jnp.sum(p, axis=-1, keepdims=True)
                w = p / denom
                ctx_buf[:, h * dh:(h + 1) * dh] = jnp.dot(
                    w, v_head[...], preferred_element_type=jnp.float32)
            out_ref[0, rows, :] += jnp.dot(
                ctx_buf[...], wo_cur[...], preferred_element_type=jnp.float32)

    return pl.pallas_call(
        body,
        grid=(N_DEV,),
        out_shape=jax.ShapeDtypeStruct((1, sq, dm), jnp.float32),
        in_specs=[
            pl.BlockSpec(memory_space=pltpu.MemorySpace.VMEM),
            pl.BlockSpec(memory_space=pltpu.MemorySpace.VMEM),
            pl.BlockSpec(memory_space=pl.ANY),
            pl.BlockSpec(memory_space=pl.ANY),
            pl.BlockSpec(memory_space=pltpu.MemorySpace.VMEM),
        ],
        out_specs=pl.BlockSpec(memory_space=pltpu.MemorySpace.VMEM),
        scratch_shapes=[
            pltpu.MemorySpace.HBM((N_DEV, dm, dq), jnp.float32),
            pltpu.MemorySpace.HBM((N_DEV, dq, dm), jnp.float32),
            pltpu.MemorySpace.VMEM((dm, dq), jnp.float32),
            pltpu.MemorySpace.VMEM((dq, dm), jnp.float32),
            pltpu.MemorySpace.VMEM((skv, dh), jnp.float32),
            pltpu.MemorySpace.VMEM((skv, dh), jnp.float32),
            pltpu.MemorySpace.VMEM((QT, dq), jnp.float32),
            pltpu.MemorySpace.VMEM((sq, skv), jnp.bfloat16),
            pltpu.SemaphoreType.DMA((4,)),
            pltpu.SemaphoreType.DMA((N_DEV - 1,)),
            pltpu.SemaphoreType.DMA((N_DEV - 1,)),
            pltpu.SemaphoreType.DMA((N_DEV - 1,)),
            pltpu.SemaphoreType.DMA((N_DEV - 1,)),
        ],
        compiler_params=pltpu.CompilerParams(
            dimension_semantics=("arbitrary",),
            collective_id=0,
        ),
    )(x, Wq, K_ext, V_ext, Wo)


# baseline (device time: 791385 ns/iter reference)
import jax
import jax.numpy as jnp
from jax import lax
from jax.experimental import pallas as pl
from jax.experimental.pallas import tpu as pltpu

N_DEV = 4
BLK = 64
QT = 256
SCALE = 0.08838834764831843


def kernel(x, Wq, K_ext, V_ext, Wo):
    _, sq, dm = x.shape
    dm_, dq = Wq.shape
    _, skv, h_tot, dh = K_ext.shape
    hg = dq // dh
    n_qt = sq // QT

    def body(x_ref, wq_ref, k_ref, v_ref, wo_ref,
             out_ref, wq_all, wo_all,
             wq_cur, wo_cur, k_head, v_head,
             ctx_buf, bias_ref, copy_sems,
             wq_ssem, wq_rsem, wo_ssem, wo_rsem):
        s = pl.program_id(0)
        my = lax.axis_index("i")
        right = lax.rem(my + 1, N_DEV)
        left = lax.rem(my + N_DEV - 1, N_DEV)
        slot = lax.rem(my + N_DEV - s, N_DEV)
        prev_h = jnp.maximum(s - 1, 0)
        prev_send_slot = lax.rem(my + N_DEV + 1 - s, N_DEV)

        @pl.when(s == 0)
        def _init():
            barrier = pltpu.get_barrier_semaphore()
            for nbr in (left, right):
                pl.semaphore_signal(barrier, inc=1, device_id=(nbr,),
                                    device_id_type=pl.DeviceIdType.MESH)
            pl.semaphore_wait(barrier, 2)
            cp_q = pltpu.make_async_copy(wq_ref, wq_all.at[my], copy_sems.at[0])
            cp_o = pltpu.make_async_copy(wo_ref, wo_all.at[my], copy_sems.at[1])
            cp_q.start()
            cp_o.start()
            for qt in range(n_qt):
                r = lax.broadcasted_iota(jnp.int32, (QT, skv), 0)
                c = lax.broadcasted_iota(jnp.int32, (QT, skv), 1)
                qb = (my * sq + qt * QT + r) // BLK
                kb = c // BLK
                keep = (qb == kb) | (kb == 0) | (lax.rem(qb + kb, 3) == 0)
                bias_ref[qt * QT:(qt + 1) * QT, :] = jnp.where(
                    keep, 0.0, -1e9).astype(jnp.bfloat16)
            cp_q.wait()
            cp_o.wait()
            out_ref[...] = jnp.zeros(out_ref.shape, out_ref.dtype)

        @pl.when(s > 0)
        def _wait_prev_hop():
            for slab, ssem, rsem in ((wq_all, wq_ssem, wq_rsem),
                                     (wo_all, wo_ssem, wo_rsem)):
                pltpu.make_async_remote_copy(
                    src_ref=slab.at[prev_send_slot],
                    dst_ref=slab.at[slot],
                    send_sem=ssem.at[prev_h],
                    recv_sem=rsem.at[prev_h],
                    device_id=(right,),
                    device_id_type=pl.DeviceIdType.MESH,
                ).wait()

        @pl.when(s < N_DEV - 1)
        def _send_hop():
            for slab, ssem, rsem in ((wq_all, wq_ssem, wq_rsem),
                                     (wo_all, wo_ssem, wo_rsem)):
                pltpu.make_async_remote_copy(
                    src_ref=slab.at[slot],
                    dst_ref=slab.at[slot],
                    send_sem=ssem.at[s],
                    recv_sem=rsem.at[s],
                    device_id=(right,),
                    device_id_type=pl.DeviceIdType.MESH,
                ).start()

        ld_q = pltpu.make_async_copy(wq_all.at[slot], wq_cur, copy_sems.at[0])
        ld_o = pltpu.make_async_copy(wo_all.at[slot], wo_cur, copy_sems.at[1])
        ld_q.start()
        ld_o.start()
        ld_q.wait()
        ld_o.wait()

        head0 = slot * hg
        for qt in range(n_qt):
            rows = pl.ds(qt * QT, QT)
            q = jnp.dot(x_ref[0, rows, :], wq_cur[...],
                        preferred_element_type=jnp.float32) * SCALE
            bias = bias_ref[rows, :].astype(jnp.float32)
            for h in range(hg):
                ck = pltpu.make_async_copy(
                    k_ref.at[0, :, head0 + h, :], k_head, copy_sems.at[2])
                cv = pltpu.make_async_copy(
                    v_ref.at[0, :, head0 + h, :], v_head, copy_sems.at[3])
                ck.start()
                cv.start()
                ck.wait()
                cv.wait()
                sc = lax.dot_general(
                    q[:, h * dh:(h + 1) * dh], k_head[...],
                    (((1,), (1,)), ((), ())),
                    preferred_element_type=jnp.float32)
                p = jnp.exp(sc + bias)
                denom = jnp.sum(p, axis=-1, keepdims=True)
                w = p / denom
                ctx_buf[:, h * dh:(h + 1) * dh] = jnp.dot(
                    w, v_head[...], preferred_element_type=jnp.float32)
            out_ref[0, rows, :] += jnp.dot(
                ctx_buf[...], wo_cur[...], preferred_element_type=jnp.float32)

    out = pl.pallas_call(
        body,
        grid=(N_DEV,),
        out_shape=(
            jax.ShapeDtypeStruct((1, sq, dm), jnp.float32),
            jax.ShapeDtypeStruct((N_DEV, dm, dq), jnp.float32),
            jax.ShapeDtypeStruct((N_DEV, dq, dm), jnp.float32),
        ),
        in_specs=[
            pl.BlockSpec(memory_space=pltpu.MemorySpace.VMEM),
            pl.BlockSpec(memory_space=pltpu.MemorySpace.VMEM),
            pl.BlockSpec(memory_space=pl.ANY),
            pl.BlockSpec(memory_space=pl.ANY),
            pl.BlockSpec(memory_space=pltpu.MemorySpace.VMEM),
        ],
        out_specs=(
            pl.BlockSpec(memory_space=pltpu.MemorySpace.VMEM),
            pl.BlockSpec(memory_space=pl.ANY),
            pl.BlockSpec(memory_space=pl.ANY),
        ),
        scratch_shapes=[
            pltpu.MemorySpace.VMEM((dm, dq), jnp.float32),
            pltpu.MemorySpace.VMEM((dq, dm), jnp.float32),
            pltpu.MemorySpace.VMEM((skv, dh), jnp.float32),
            pltpu.MemorySpace.VMEM((skv, dh), jnp.float32),
            pltpu.MemorySpace.VMEM((QT, dq), jnp.float32),
            pltpu.MemorySpace.VMEM((sq, skv), jnp.bfloat16),
            pltpu.SemaphoreType.DMA((4,)),
            pltpu.SemaphoreType.DMA((N_DEV - 1,)),
            pltpu.SemaphoreType.DMA((N_DEV - 1,)),
            pltpu.SemaphoreType.DMA((N_DEV - 1,)),
            pltpu.SemaphoreType.DMA((N_DEV - 1,)),
        ],
        compiler_params=pltpu.CompilerParams(
            dimension_semantics=("arbitrary",),
            collective_id=0,
            vmem_limit_bytes=64 * 1024 * 1024,
        ),
    )(x, Wq, K_ext, V_ext, Wo)
    return out[0]


# device time: 442123 ns/iter; 1.7900x vs baseline; 1.7900x over previous
import jax
import jax.numpy as jnp
from jax import lax
from jax.experimental import pallas as pl
from jax.experimental.pallas import tpu as pltpu

N_DEV = 4
BLK = 64
QT = 256
SCALE = 0.08838834764831843


def kernel(x, Wq, K_ext, V_ext, Wo):
    _, sq, dm = x.shape
    dm_, dq = Wq.shape
    _, skv, h_tot, dh = K_ext.shape
    hg = dq // dh
    n_qt = sq // QT

    def body(x_ref, wq_ref, k_ref, v_ref, wo_ref,
             out_ref, wq_all, wo_all,
             wq_cur, wo_cur, k_head, v_head,
             ctx_buf, bias_ref, copy_sems,
             wq_ssem, wq_rsem, wo_ssem, wo_rsem):
        s = pl.program_id(0)
        my = lax.axis_index("i")
        right = lax.rem(my + 1, N_DEV)
        left = lax.rem(my + N_DEV - 1, N_DEV)
        slot = lax.rem(my + N_DEV - s, N_DEV)
        prev_h = jnp.maximum(s - 1, 0)
        prev_send_slot = lax.rem(my + N_DEV + 1 - s, N_DEV)

        @pl.when(s == 0)
        def _init():
            barrier = pltpu.get_barrier_semaphore()
            for nbr in (left, right):
                pl.semaphore_signal(barrier, inc=1, device_id=(nbr,),
                                    device_id_type=pl.DeviceIdType.MESH)
            pl.semaphore_wait(barrier, 2)
            cp_q = pltpu.make_async_copy(wq_ref, wq_all.at[my], copy_sems.at[0])
            cp_o = pltpu.make_async_copy(wo_ref, wo_all.at[my], copy_sems.at[1])
            cp_q.start()
            cp_o.start()
            for qt in range(n_qt):
                r = lax.broadcasted_iota(jnp.int32, (QT, skv), 0)
                c = lax.broadcasted_iota(jnp.int32, (QT, skv), 1)
                qb = (my * sq + qt * QT + r) // BLK
                kb = c // BLK
                keep = (qb == kb) | (kb == 0) | (lax.rem(qb + kb, 3) == 0)
                bias_ref[qt * QT:(qt + 1) * QT, :] = jnp.where(
                    keep, 0.0, -1e9).astype(jnp.bfloat16)
            cp_q.wait()
            cp_o.wait()
            out_ref[...] = jnp.zeros(out_ref.shape, out_ref.dtype)

        @pl.when(s > 0)
        def _wait_prev_hop():
            for slab, ssem, rsem in ((wq_all, wq_ssem, wq_rsem),
                                     (wo_all, wo_ssem, wo_rsem)):
                pltpu.make_async_remote_copy(
                    src_ref=slab.at[prev_send_slot],
                    dst_ref=slab.at[slot],
                    send_sem=ssem.at[prev_h],
                    recv_sem=rsem.at[prev_h],
                    device_id=(right,),
                    device_id_type=pl.DeviceIdType.MESH,
                ).wait()

        @pl.when(s < N_DEV - 1)
        def _send_hop():
            for slab, ssem, rsem in ((wq_all, wq_ssem, wq_rsem),
                                     (wo_all, wo_ssem, wo_rsem)):
                pltpu.make_async_remote_copy(
                    src_ref=slab.at[slot],
                    dst_ref=slab.at[slot],
                    send_sem=ssem.at[s],
                    recv_sem=rsem.at[s],
                    device_id=(right,),
                    device_id_type=pl.DeviceIdType.MESH,
                ).start()

        ld_q = pltpu.make_async_copy(wq_all.at[slot], wq_cur, copy_sems.at[0])
        ld_o = pltpu.make_async_copy(wo_all.at[slot], wo_cur, copy_sems.at[1])
        ld_q.start()
        ld_o.start()
        ld_q.wait()
        ld_o.wait()

        head0 = slot * hg
        pairs = [(qt, h) for qt in range(n_qt) for h in range(hg)]

        def start_kv(i):
            _, h = pairs[i]
            sl = i % 2
            pltpu.make_async_copy(
                k_ref.at[0, :, head0 + h, :], k_head.at[sl],
                copy_sems.at[2 + sl]).start()
            pltpu.make_async_copy(
                v_ref.at[0, :, head0 + h, :], v_head.at[sl],
                copy_sems.at[4 + sl]).start()

        def wait_kv(i):
            _, h = pairs[i]
            sl = i % 2
            pltpu.make_async_copy(
                k_ref.at[0, :, head0 + h, :], k_head.at[sl],
                copy_sems.at[2 + sl]).wait()
            pltpu.make_async_copy(
                v_ref.at[0, :, head0 + h, :], v_head.at[sl],
                copy_sems.at[4 + sl]).wait()

        start_kv(0)
        q = None
        for i, (qt, h) in enumerate(pairs):
            if i + 1 < len(pairs):
                start_kv(i + 1)
            rows = pl.ds(qt * QT, QT)
            if h == 0:
                q = jnp.dot(x_ref[0, rows, :], wq_cur[...],
                            preferred_element_type=jnp.float32) * SCALE
                bias = bias_ref[rows, :].astype(jnp.float32)
            wait_kv(i)
            sl = i % 2
            sc = lax.dot_general(
                q[:, h * dh:(h + 1) * dh], k_head[sl],
                (((1,), (1,)), ((), ())),
                preferred_element_type=jnp.float32)
            p = jnp.exp(sc + bias)
            denom = jnp.sum(p, axis=-1, keepdims=True)
            ctx_buf[:, h * dh:(h + 1) * dh] = jnp.dot(
                p, v_head[sl],
                preferred_element_type=jnp.float32) / denom
            if h == hg - 1:
                out_ref[0, rows, :] += jnp.dot(
                    ctx_buf[...], wo_cur[...],
                    preferred_element_type=jnp.float32)

    out = pl.pallas_call(
        body,
        grid=(N_DEV,),
        out_shape=(
            jax.ShapeDtypeStruct((1, sq, dm), jnp.float32),
            jax.ShapeDtypeStruct((N_DEV, dm, dq), jnp.float32),
            jax.ShapeDtypeStruct((N_DEV, dq, dm), jnp.float32),
        ),
        in_specs=[
            pl.BlockSpec(memory_space=pltpu.MemorySpace.VMEM),
            pl.BlockSpec(memory_space=pltpu.MemorySpace.VMEM),
            pl.BlockSpec(memory_space=pl.ANY),
            pl.BlockSpec(memory_space=pl.ANY),
            pl.BlockSpec(memory_space=pltpu.MemorySpace.VMEM),
        ],
        out_specs=(
            pl.BlockSpec(memory_space=pltpu.MemorySpace.VMEM),
            pl.BlockSpec(memory_space=pl.ANY),
            pl.BlockSpec(memory_space=pl.ANY),
        ),
        scratch_shapes=[
            pltpu.MemorySpace.VMEM((dm, dq), jnp.float32),
            pltpu.MemorySpace.VMEM((dq, dm), jnp.float32),
            pltpu.MemorySpace.VMEM((2, skv, dh), jnp.float32),
            pltpu.MemorySpace.VMEM((2, skv, dh), jnp.float32),
            pltpu.MemorySpace.VMEM((QT, dq), jnp.float32),
            pltpu.MemorySpace.VMEM((sq, skv), jnp.bfloat16),
            pltpu.SemaphoreType.DMA((6,)),
            pltpu.SemaphoreType.DMA((N_DEV - 1,)),
            pltpu.SemaphoreType.DMA((N_DEV - 1,)),
            pltpu.SemaphoreType.DMA((N_DEV - 1,)),
            pltpu.SemaphoreType.DMA((N_DEV - 1,)),
        ],
        compiler_params=pltpu.CompilerParams(
            dimension_semantics=("arbitrary",),
            collective_id=0,
            vmem_limit_bytes=64 * 1024 * 1024,
        ),
    )(x, Wq, K_ext, V_ext, Wo)
    return out[0]
